# baseline (speedup 1.0000x reference)
import functools

import jax
import jax.numpy as jnp
from jax.experimental import pallas as pl
from jax.experimental.pallas import tpu as pltpu

N = 8192
K = 32
D = 7
BR = 256
BISECT_ITERS = 18


def _body(xf_ref, xft_ref, w_ref, a_ref, q_ref, y_ref):
    a = a_ref[...]
    fw = jax.nn.softmax(a, axis=1)
    xf = xf_ref[...]
    out = jnp.dot(xf, w_ref[...].T, preferred_element_type=jnp.float32)
    out = jnp.clip(out * fw, -1.0, 1.0)

    kt = xft_ref[...]
    kn2 = jnp.sum(kt * kt, axis=0, keepdims=True)
    kt_n = kt * jax.lax.rsqrt(kn2)

    q = q_ref[...]
    qn2 = jnp.sum(q * q, axis=1, keepdims=True)
    q_n = q * jax.lax.rsqrt(qn2)

    s = jax.lax.dot_general(
        q_n, kt_n, (((1,), (0,)), ((), ())),
        preferred_element_type=jnp.float32,
    )

    kf = jnp.float32(K)
    g = jnp.maximum(s[:, : N // 2], s[:, N // 2:])
    g = jnp.maximum(g[:, : N // 4], g[:, N // 4:])
    g = jnp.maximum(g[:, : N // 8], g[:, N // 8:])
    g = jnp.maximum(g[:, : N // 16], g[:, N // 16:])

    ones_b = jnp.ones((N, 128), jnp.bfloat16)

    def count_ge(v, t):
        m = jnp.where(v >= t, 1.0, 0.0).astype(jnp.bfloat16)
        c = jax.lax.dot_general(
            m, ones_b[: v.shape[1]], (((1,), (0,)), ((), ())),
            preferred_element_type=jnp.float32,
        )
        return c[:, :1]

    def gbisect(_, carry):
        lo_a, hi_a, lo_b, hi_b = carry
        mid_a = 0.5 * (lo_a + hi_a)
        mid_b = 0.5 * (lo_b + hi_b)
        ge_a = count_ge(g, mid_a) >= kf
        ge_b = count_ge(g, mid_b) >= 2.0
        return (jnp.where(ge_a, mid_a, lo_a), jnp.where(ge_a, hi_a, mid_a),
                jnp.where(ge_b, mid_b, lo_b), jnp.where(ge_b, hi_b, mid_b))

    f11 = jnp.full((BR, 1), 1.1, jnp.float32)
    lo_a, _, _, hi_b = jax.lax.fori_loop(
        0, 20, gbisect, (-f11, f11, -f11, f11))

    def bisect(_, carry):
        lo, hi = carry
        mid = 0.5 * (lo + hi)
        ge = count_ge(s, mid) >= kf
        return jnp.where(ge, mid, lo), jnp.where(ge, hi, mid)

    lo, _ = jax.lax.fori_loop(0, BISECT_ITERS, bisect, (lo_a, hi_b))

    w = jnp.where(s >= lo, jnp.exp(s), 0.0)
    z = jnp.sum(w, axis=1, keepdims=True)
    y = jnp.dot(w, out, preferred_element_type=jnp.float32)
    y_ref[...] = y / z


@jax.jit
def kernel(x, weight, a):
    xf = x[:, :D]
    xft = xf.T
    a2 = a.reshape(1, D)
    grid = N // BR
    y = pl.pallas_call(
        _body,
        grid=(grid,),
        in_specs=[
            pl.BlockSpec((N, D), lambda i: (0, 0)),
            pl.BlockSpec((D, N), lambda i: (0, 0)),
            pl.BlockSpec((D, D), lambda i: (0, 0)),
            pl.BlockSpec((1, D), lambda i: (0, 0)),
            pl.BlockSpec((BR, D), lambda i: (i, 0)),
        ],
        out_specs=pl.BlockSpec((BR, D), lambda i: (i, 0)),
        out_shape=jax.ShapeDtypeStruct((N, D), jnp.float32),
        compiler_params=pltpu.CompilerParams(
            dimension_semantics=("arbitrary",),
        ),
    )(xf, xft, weight, a2, xf)
    return y


# speedup vs baseline: 1.2837x; 1.2837x over previous
import functools

import jax
import jax.numpy as jnp
from jax.experimental import pallas as pl
from jax.experimental.pallas import tpu as pltpu

N = 8192
K = 32
D = 7
BR = 256
BISECT_ITERS = 18


def _body(xf_ref, xft_ref, w_ref, a_ref, q_ref, y_ref):
    a = a_ref[...]
    fw = jax.nn.softmax(a, axis=1)
    xf = xf_ref[...]
    out = jnp.dot(xf, w_ref[...].T, preferred_element_type=jnp.float32)
    out = jnp.clip(out * fw, -1.0, 1.0)

    kt = xft_ref[...]
    kn2 = jnp.sum(kt * kt, axis=0, keepdims=True)
    kt_n = kt * jax.lax.rsqrt(kn2)

    q = q_ref[...]
    qn2 = jnp.sum(q * q, axis=1, keepdims=True)
    q_n = q * jax.lax.rsqrt(qn2)

    s = jax.lax.dot_general(
        q_n, kt_n, (((1,), (0,)), ((), ())),
        preferred_element_type=jnp.float32,
    )

    kf = jnp.float32(K)
    g = jnp.maximum(s[:, : N // 2], s[:, N // 2:])
    g = jnp.maximum(g[:, : N // 4], g[:, N // 4:])
    g = jnp.maximum(g[:, : N // 8], g[:, N // 8:])
    g = jnp.maximum(g[:, : N // 16], g[:, N // 16:])

    def count_ge(v, t):
        return jnp.sum((v >= t).astype(jnp.float32), axis=1, keepdims=True)

    def gbisect(_, carry):
        lo_a, hi_a, lo_b, hi_b = carry
        mid_a = 0.5 * (lo_a + hi_a)
        mid_b = 0.5 * (lo_b + hi_b)
        ge_a = count_ge(g, mid_a) >= kf
        ge_b = count_ge(g, mid_b) >= 2.0
        return (jnp.where(ge_a, mid_a, lo_a), jnp.where(ge_a, hi_a, mid_a),
                jnp.where(ge_b, mid_b, lo_b), jnp.where(ge_b, hi_b, mid_b))

    f11 = jnp.full((BR, 1), 1.1, jnp.float32)
    lo_a, _, _, hi_b = jax.lax.fori_loop(
        0, 20, gbisect, (-f11, f11, -f11, f11))

    def bisect(_, carry):
        lo, hi = carry
        mid = 0.5 * (lo + hi)
        ge = count_ge(s, mid) >= kf
        return jnp.where(ge, mid, lo), jnp.where(ge, hi, mid)

    lo, _ = jax.lax.fori_loop(0, BISECT_ITERS, bisect, (lo_a, hi_b))

    w = jnp.where(s >= lo, jnp.exp(s), 0.0)
    z = jnp.sum(w, axis=1, keepdims=True)
    y = jnp.dot(w, out, preferred_element_type=jnp.float32)
    y_ref[...] = y / z


@jax.jit
def kernel(x, weight, a):
    xf = x[:, :D]
    xft = xf.T
    a2 = a.reshape(1, D)
    grid = N // BR
    y = pl.pallas_call(
        _body,
        grid=(grid,),
        in_specs=[
            pl.BlockSpec((N, D), lambda i: (0, 0)),
            pl.BlockSpec((D, N), lambda i: (0, 0)),
            pl.BlockSpec((D, D), lambda i: (0, 0)),
            pl.BlockSpec((1, D), lambda i: (0, 0)),
            pl.BlockSpec((BR, D), lambda i: (i, 0)),
        ],
        out_specs=pl.BlockSpec((BR, D), lambda i: (i, 0)),
        out_shape=jax.ShapeDtypeStruct((N, D), jnp.float32),
        compiler_params=pltpu.CompilerParams(
            dimension_semantics=("arbitrary",),
        ),
    )(xf, xft, weight, a2, xf)
    return y


# speedup vs baseline: 1.6455x; 1.2819x over previous
import functools

import jax
import jax.numpy as jnp
from jax.experimental import pallas as pl
from jax.experimental.pallas import tpu as pltpu

N = 8192
K = 32
D = 7
BR = 256
BISECT_ITERS = 13
G_ITERS = 16


def _body(xf_ref, xft_ref, w_ref, a_ref, q_ref, y_ref):
    a = a_ref[...]
    fw = jax.nn.softmax(a, axis=1)
    xf = xf_ref[...]
    out = jnp.dot(xf, w_ref[...].T, preferred_element_type=jnp.float32)
    out = jnp.clip(out * fw, -1.0, 1.0)

    kt = xft_ref[...]
    kn2 = jnp.sum(kt * kt, axis=0, keepdims=True)
    kt_n = kt * jax.lax.rsqrt(kn2)

    q = q_ref[...]
    qn2 = jnp.sum(q * q, axis=1, keepdims=True)
    q_n = q * jax.lax.rsqrt(qn2)

    s = jax.lax.dot_general(
        q_n, kt_n, (((1,), (0,)), ((), ())),
        preferred_element_type=jnp.float32,
    )

    kf = jnp.float32(K)
    g = jnp.maximum(s[:, : N // 2], s[:, N // 2:])
    g = jnp.maximum(g[:, : N // 4], g[:, N // 4:])
    g = jnp.maximum(g[:, : N // 8], g[:, N // 8:])
    g = jnp.maximum(g[:, : N // 16], g[:, N // 16:])

    def count_ge(v, t):
        return jnp.sum((v >= t).astype(jnp.float32), axis=1, keepdims=True)

    def gbisect(_, carry):
        lo_a, hi_a, lo_b, hi_b = carry
        mid_a = 0.5 * (lo_a + hi_a)
        mid_b = 0.5 * (lo_b + hi_b)
        ge_a = count_ge(g, mid_a) >= kf
        ge_b = count_ge(g, mid_b) >= 2.0
        return (jnp.where(ge_a, mid_a, lo_a), jnp.where(ge_a, hi_a, mid_a),
                jnp.where(ge_b, mid_b, lo_b), jnp.where(ge_b, hi_b, mid_b))

    f11 = jnp.full((BR, 1), 1.1, jnp.float32)
    lo_a, _, _, hi_b = jax.lax.fori_loop(
        0, G_ITERS, gbisect, (-f11, f11, -f11, f11))

    def bisect(_, carry):
        lo, hi = carry
        mid = 0.5 * (lo + hi)
        ge = count_ge(s, mid) >= kf
        return jnp.where(ge, mid, lo), jnp.where(ge, hi, mid)

    lo, _ = jax.lax.fori_loop(0, BISECT_ITERS, bisect, (lo_a, hi_b))

    w = jnp.where(s >= lo, jnp.exp(s), 0.0)
    out1 = jnp.concatenate([out, jnp.ones((N, 1), jnp.float32)], axis=1)
    yz = jax.lax.dot_general(
        w, out1, (((1,), (0,)), ((), ())),
        preferred_element_type=jnp.float32,
    )
    y_ref[...] = yz[:, :D] / yz[:, D:]


@jax.jit
def kernel(x, weight, a):
    xf = x[:, :D]
    xft = xf.T
    a2 = a.reshape(1, D)
    grid = N // BR
    y = pl.pallas_call(
        _body,
        grid=(grid,),
        in_specs=[
            pl.BlockSpec((N, D), lambda i: (0, 0)),
            pl.BlockSpec((D, N), lambda i: (0, 0)),
            pl.BlockSpec((D, D), lambda i: (0, 0)),
            pl.BlockSpec((1, D), lambda i: (0, 0)),
            pl.BlockSpec((BR, D), lambda i: (i, 0)),
        ],
        out_specs=pl.BlockSpec((BR, D), lambda i: (i, 0)),
        out_shape=jax.ShapeDtypeStruct((N, D), jnp.float32),
        compiler_params=pltpu.CompilerParams(
            dimension_semantics=("arbitrary",),
        ),
    )(xf, xft, weight, a2, xf)
    return y


# speedup vs baseline: 2.7169x; 1.6511x over previous
import functools

import jax
import jax.numpy as jnp
from jax.experimental import pallas as pl
from jax.experimental.pallas import tpu as pltpu

N = 8192
K = 32
D = 7
BR = 256
BISECT_ITERS = 24


def _body(xf_ref, xft_ref, w_ref, a_ref, q_ref, y_ref):
    a = a_ref[...]
    fw = jax.nn.softmax(a, axis=1)
    xf = xf_ref[...]
    out = jnp.dot(xf, w_ref[...].T, preferred_element_type=jnp.float32)
    out = jnp.clip(out * fw, -1.0, 1.0)

    kt = xft_ref[...]
    kn2 = jnp.sum(kt * kt, axis=0, keepdims=True)
    kt_n = kt * jax.lax.rsqrt(kn2)

    q = q_ref[...]
    qn2 = jnp.sum(q * q, axis=1, keepdims=True)
    q_n = q * jax.lax.rsqrt(qn2)

    s = jax.lax.dot_general(
        q_n, kt_n, (((1,), (0,)), ((), ())),
        preferred_element_type=jnp.float32,
    )

    kf = jnp.float32(K)
    W = N // 16
    sl = [s[:, r * W:(r + 1) * W] for r in range(16)]

    def merge3(A, B):
        a1, a2, a3 = A
        b1, b2, b3 = B
        c1 = jnp.maximum(a1, b1)
        l1 = jnp.minimum(a1, b1)
        h2 = jnp.maximum(a2, b2)
        c2 = jnp.maximum(l1, h2)
        c3 = jnp.maximum(jnp.minimum(l1, h2), jnp.maximum(a3, b3))
        return (c1, c2, c3)

    pairs = [(jnp.maximum(sl[2 * k], sl[2 * k + 1]),
              jnp.minimum(sl[2 * k], sl[2 * k + 1])) for k in range(8)]
    quads = []
    for k in range(4):
        (a1, a2), (b1, b2) = pairs[2 * k], pairs[2 * k + 1]
        h1 = jnp.maximum(a1, b1)
        l1 = jnp.minimum(a1, b1)
        h2 = jnp.maximum(a2, b2)
        quads.append((h1, jnp.maximum(l1, h2), jnp.minimum(l1, h2)))
    m1, m2, m3 = merge3(merge3(quads[0], quads[1]), merge3(quads[2], quads[3]))
    surr = jnp.concatenate([m1, m2, m3], axis=1)

    def count_ge(v, t):
        return jnp.sum((v >= t).astype(jnp.float32), axis=1, keepdims=True)

    def bisect(_, carry):
        lo, hi = carry
        mid = 0.5 * (lo + hi)
        ge = count_ge(surr, mid) >= kf
        return jnp.where(ge, mid, lo), jnp.where(ge, hi, mid)

    f102 = jnp.full((BR, 1), 1.02, jnp.float32)
    lo, _ = jax.lax.fori_loop(0, BISECT_ITERS, bisect, (-f102, f102))

    w = jnp.where(s >= lo, jnp.exp(s), 0.0)
    out1 = jnp.concatenate([out, jnp.ones((N, 1), jnp.float32)], axis=1)
    yz = jax.lax.dot_general(
        w, out1, (((1,), (0,)), ((), ())),
        preferred_element_type=jnp.float32,
    )
    y_ref[...] = yz[:, :D] / yz[:, D:]


@jax.jit
def kernel(x, weight, a):
    xf = x[:, :D]
    xft = xf.T
    a2 = a.reshape(1, D)
    grid = N // BR
    y = pl.pallas_call(
        _body,
        grid=(grid,),
        in_specs=[
            pl.BlockSpec((N, D), lambda i: (0, 0)),
            pl.BlockSpec((D, N), lambda i: (0, 0)),
            pl.BlockSpec((D, D), lambda i: (0, 0)),
            pl.BlockSpec((1, D), lambda i: (0, 0)),
            pl.BlockSpec((BR, D), lambda i: (i, 0)),
        ],
        out_specs=pl.BlockSpec((BR, D), lambda i: (i, 0)),
        out_shape=jax.ShapeDtypeStruct((N, D), jnp.float32),
        compiler_params=pltpu.CompilerParams(
            dimension_semantics=("arbitrary",),
        ),
    )(xf, xft, weight, a2, xf)
    return y


# speedup vs baseline: 3.0793x; 1.1334x over previous
import functools

import jax
import jax.numpy as jnp
from jax.experimental import pallas as pl
from jax.experimental.pallas import tpu as pltpu

N = 8192
K = 32
D = 7
BR = 512
BISECT_ITERS = 24


def _body(xft_ref, w_ref, a_ref, q_ref, y_ref, out1_ref, ktn_ref):
    @pl.when(pl.program_id(0) == 0)
    def _prologue():
        kt = xft_ref[...]
        kn2 = jnp.sum(kt * kt, axis=0, keepdims=True)
        ktn_ref[...] = kt * jax.lax.rsqrt(kn2)
        fw = jax.nn.softmax(a_ref[...], axis=1)
        out = jax.lax.dot_general(
            kt, w_ref[...], (((0,), (1,)), ((), ())),
            preferred_element_type=jnp.float32,
        )
        out = jnp.clip(out * fw, -1.0, 1.0)
        out1_ref[...] = jnp.concatenate(
            [out, jnp.ones((N, 1), jnp.float32)], axis=1)

    q = q_ref[...]
    qn2 = jnp.sum(q * q, axis=1, keepdims=True)
    q_n = q * jax.lax.rsqrt(qn2)

    s = jax.lax.dot_general(
        q_n, ktn_ref[...], (((1,), (0,)), ((), ())),
        preferred_element_type=jnp.float32,
    )

    kf = jnp.float32(K)
    W = N // 16
    sl = [s[:, r * W:(r + 1) * W] for r in range(16)]

    def merge3(A, B):
        a1, a2, a3 = A
        b1, b2, b3 = B
        c1 = jnp.maximum(a1, b1)
        l1 = jnp.minimum(a1, b1)
        h2 = jnp.maximum(a2, b2)
        c2 = jnp.maximum(l1, h2)
        c3 = jnp.maximum(jnp.minimum(l1, h2), jnp.maximum(a3, b3))
        return (c1, c2, c3)

    pairs = [(jnp.maximum(sl[2 * k], sl[2 * k + 1]),
              jnp.minimum(sl[2 * k], sl[2 * k + 1])) for k in range(8)]
    quads = []
    for k in range(4):
        (a1, a2), (b1, b2) = pairs[2 * k], pairs[2 * k + 1]
        h1 = jnp.maximum(a1, b1)
        l1 = jnp.minimum(a1, b1)
        h2 = jnp.maximum(a2, b2)
        quads.append((h1, jnp.maximum(l1, h2), jnp.minimum(l1, h2)))
    m1, m2, m3 = merge3(merge3(quads[0], quads[1]), merge3(quads[2], quads[3]))
    surr = jnp.concatenate([m1, m2, m3], axis=1)

    def count_ge(v, t):
        return jnp.sum((v >= t).astype(jnp.float32), axis=1, keepdims=True)

    def bisect(_, carry):
        lo, hi = carry
        mid = 0.5 * (lo + hi)
        ge = count_ge(surr, mid) >= kf
        return jnp.where(ge, mid, lo), jnp.where(ge, hi, mid)

    f102 = jnp.full((BR, 1), 1.02, jnp.float32)
    lo, _ = jax.lax.fori_loop(0, BISECT_ITERS, bisect, (-f102, f102))

    w = jnp.where(s >= lo, jnp.exp(s), 0.0)
    yz = jax.lax.dot_general(
        w, out1_ref[...], (((1,), (0,)), ((), ())),
        preferred_element_type=jnp.float32,
    )
    y_ref[...] = yz[:, :D] / yz[:, D:]


@jax.jit
def kernel(x, weight, a):
    xf = x[:, :D]
    xft = xf.T
    a2 = a.reshape(1, D)
    grid = N // BR
    y = pl.pallas_call(
        _body,
        grid=(grid,),
        in_specs=[
            pl.BlockSpec((D, N), lambda i: (0, 0)),
            pl.BlockSpec((D, D), lambda i: (0, 0)),
            pl.BlockSpec((1, D), lambda i: (0, 0)),
            pl.BlockSpec((BR, D), lambda i: (i, 0)),
        ],
        out_specs=pl.BlockSpec((BR, D), lambda i: (i, 0)),
        out_shape=jax.ShapeDtypeStruct((N, D), jnp.float32),
        scratch_shapes=[
            pltpu.VMEM((N, D + 1), jnp.float32),
            pltpu.VMEM((D, N), jnp.float32),
        ],
        compiler_params=pltpu.CompilerParams(
            dimension_semantics=("arbitrary",),
        ),
    )(xft, weight, a2, xf)
    return y


# speedup vs baseline: 4.0759x; 1.3236x over previous
import functools

import jax
import jax.numpy as jnp
from jax.experimental import pallas as pl
from jax.experimental.pallas import tpu as pltpu

N = 8192
K = 32
D = 7
BR = 512
BISECT_ITERS = 17


def _body(xft_ref, w_ref, a_ref, q_ref, y_ref, out1_ref, ktn_ref):
    @pl.when(pl.program_id(0) == 0)
    def _prologue():
        kt = xft_ref[...]
        kn2 = jnp.sum(kt * kt, axis=0, keepdims=True)
        ktn_ref[...] = kt * jax.lax.rsqrt(kn2)
        fw = jax.nn.softmax(a_ref[...], axis=1)
        out = jax.lax.dot_general(
            kt, w_ref[...], (((0,), (1,)), ((), ())),
            preferred_element_type=jnp.float32,
        )
        out = jnp.clip(out * fw, -1.0, 1.0)
        out1_ref[...] = jnp.concatenate(
            [out, jnp.ones((N, 1), jnp.float32)], axis=1)

    q = q_ref[...]
    qn2 = jnp.sum(q * q, axis=1, keepdims=True)
    q_n = q * jax.lax.rsqrt(qn2)

    s = jax.lax.dot_general(
        q_n, ktn_ref[...], (((1,), (0,)), ((), ())),
        preferred_element_type=jnp.float32,
    )

    kf = jnp.float32(K)
    W = N // 32
    sl = [s[:, r * W:(r + 1) * W] for r in range(32)]
    mx, mn = jnp.maximum, jnp.minimum

    def merge22(A, B):
        a1, a2 = A
        b1, b2 = B
        l1 = mn(a1, b1)
        return (mx(a1, b1), mx(l1, mx(a2, b2)),
                mx(mn(a2, b1), mn(a1, b2)), mn(a2, b2))

    def merge44(A, B):
        a1, a2, a3, a4 = A
        b1, b2, b3, b4 = B
        l1 = mn(a1, b1)
        r1 = mx(a1, b1)
        r2 = mx(l1, mx(a2, b2))
        r3 = mx(mx(a3, b3), mx(mn(a2, b1), mn(a1, b2)))
        r4 = mx(mx(mx(a4, b4), mn(a3, b1)), mx(mn(a2, b2), mn(a1, b3)))
        return (r1, r2, r3, r4)

    pairs = [(mx(sl[2 * k], sl[2 * k + 1]),
              mn(sl[2 * k], sl[2 * k + 1])) for k in range(16)]
    lvl = [merge22(pairs[2 * k], pairs[2 * k + 1]) for k in range(8)]
    while len(lvl) > 1:
        lvl = [merge44(lvl[2 * k], lvl[2 * k + 1]) for k in range(len(lvl) // 2)]
    surr = jnp.concatenate(list(lvl[0]), axis=1)

    def count_ge(v, t):
        return jnp.sum((v >= t).astype(jnp.float32), axis=1, keepdims=True)

    def bisect(_, carry):
        lo, hi = carry
        mid = 0.5 * (lo + hi)
        ge = count_ge(surr, mid) >= kf
        return jnp.where(ge, mid, lo), jnp.where(ge, hi, mid)

    f102 = jnp.full((BR, 1), 1.02, jnp.float32)
    lo, _ = jax.lax.fori_loop(0, BISECT_ITERS, bisect, (-f102, f102))

    w = jnp.where(s >= lo, jnp.exp(s), 0.0)
    yz = jax.lax.dot_general(
        w, out1_ref[...], (((1,), (0,)), ((), ())),
        preferred_element_type=jnp.float32,
    )
    y_ref[...] = yz[:, :D] / yz[:, D:]


@jax.jit
def kernel(x, weight, a):
    xf = x[:, :D]
    xft = xf.T
    a2 = a.reshape(1, D)
    grid = N // BR
    y = pl.pallas_call(
        _body,
        grid=(grid,),
        in_specs=[
            pl.BlockSpec((D, N), lambda i: (0, 0)),
            pl.BlockSpec((D, D), lambda i: (0, 0)),
            pl.BlockSpec((1, D), lambda i: (0, 0)),
            pl.BlockSpec((BR, D), lambda i: (i, 0)),
        ],
        out_specs=pl.BlockSpec((BR, D), lambda i: (i, 0)),
        out_shape=jax.ShapeDtypeStruct((N, D), jnp.float32),
        scratch_shapes=[
            pltpu.VMEM((N, D + 1), jnp.float32),
            pltpu.VMEM((D, N), jnp.float32),
        ],
        compiler_params=pltpu.CompilerParams(
            dimension_semantics=("arbitrary",),
        ),
    )(xft, weight, a2, xf)
    return y
